# baseline (speedup 1.0000x reference)
import jax
import jax.numpy as jnp
from jax import lax
from jax.experimental import pallas as pl
from jax.experimental.pallas import tpu as pltpu
from jax.experimental.pallas import tpu_sc as plsc

N = 10000
E = 320000
H = 128
B = 64
OUT = 10
NP = 10112
NBLOCKS = NP // 128

NC = 2
NS = 16
LN = 16
NTILES = NC * NS
EPT = E // NTILES
G = 64
NBLK = -(-EPT // G)
RPT = 632
NS_ROWS = 128
T2W = 5008


def _lift_kernel(x_ref, w_ref, b_ref, q_ref, k_ref, v_ref, s_ref, t_ref):
    y = jnp.dot(x_ref[...], w_ref[...], preferred_element_type=jnp.float32)
    y = y + b_ref[...]
    q_ref[...] = y[:, :128]
    k_ref[...] = y[:, 128:256]
    v_ref[...] = y[:, 256:384]
    s_ref[...] = y[:, 384:512]
    t_ref[...] = y[:, 512:]


_LIFT_OUT_SPECS = [
    pl.BlockSpec((128, H), lambda i: (i, 0)),
    pl.BlockSpec((128, H), lambda i: (i, 0)),
    pl.BlockSpec((128, H), lambda i: (i, 0)),
    pl.BlockSpec((128, H), lambda i: (i, 0)),
    pl.BlockSpec((128, H), lambda i: (i, 0)),
]
_LIFT_OUT_SHAPE = [
    jax.ShapeDtypeStruct((NP, H), jnp.float32),
    jax.ShapeDtypeStruct((NP, H), jnp.float32),
    jax.ShapeDtypeStruct((NP, H), jnp.float32),
    jax.ShapeDtypeStruct((NP, H), jnp.float32),
    jax.ShapeDtypeStruct((NP, H), jnp.float32),
]


def _lift(xp, W, b):
    return pl.pallas_call(
        _lift_kernel,
        grid=(NBLOCKS,),
        in_specs=[
            pl.BlockSpec((128, H), lambda i: (i, 0)),
            pl.BlockSpec((H, 640), lambda i: (0, 0)),
            pl.BlockSpec((1, 640), lambda i: (0, 0)),
        ],
        out_specs=_LIFT_OUT_SPECS,
        out_shape=_LIFT_OUT_SHAPE,
    )(xp, W, b)


def _node_out(av, sv, s, i):
    ohi = jnp.where(
        lax.broadcasted_iota(jnp.int32, (128, 128), 0) == i, 1.0, 0.0)
    den = jnp.dot(sv, ohi, preferred_element_type=jnp.float32)
    return av / (den + 1e-16) + s


def _mid_kernel(a0_ref, a1_ref, s0_ref, s1_ref, sk_ref, w_ref, b_ref,
                q_ref, k_ref, v_ref, s2_ref, t_ref):
    i = pl.program_id(0)
    sv = s0_ref[...] + s1_ref[...]
    h = jnp.maximum(_node_out(a0_ref[...] + a1_ref[...], sv, sk_ref[...], i),
                    0.0)
    rows = i * 128 + lax.broadcasted_iota(jnp.int32, (128, 1), 0)
    h = jnp.where(rows < N, h, 0.0)
    y = jnp.dot(h, w_ref[...], preferred_element_type=jnp.float32)
    y = y + b_ref[...]
    q_ref[...] = y[:, :128]
    k_ref[...] = y[:, 128:256]
    v_ref[...] = y[:, 256:384]
    s2_ref[...] = y[:, 384:512]
    t_ref[...] = y[:, 512:]


def _mid(a0, a1, s0, s1, sk, W, b):
    return pl.pallas_call(
        _mid_kernel,
        grid=(NBLOCKS,),
        in_specs=[
            pl.BlockSpec((128, H), lambda i: (i, 0)),
            pl.BlockSpec((128, H), lambda i: (i, 0)),
            pl.BlockSpec((128, 128), lambda i: (0, 0)),
            pl.BlockSpec((128, 128), lambda i: (0, 0)),
            pl.BlockSpec((128, H), lambda i: (i, 0)),
            pl.BlockSpec((H, 640), lambda i: (0, 0)),
            pl.BlockSpec((1, 640), lambda i: (0, 0)),
        ],
        out_specs=_LIFT_OUT_SPECS,
        out_shape=_LIFT_OUT_SHAPE,
    )(a0, a1, s0, s1, sk, W, b)


def _fin_kernel(a0_ref, a1_ref, s0_ref, s1_ref, sk_ref, batch_ref,
                wlt_ref, bl_ref, out_ref, pooled_ref, cnt_ref):
    i = pl.program_id(0)

    @pl.when(i == 0)
    def _():
        pooled_ref[...] = jnp.zeros_like(pooled_ref)
        cnt_ref[...] = jnp.zeros_like(cnt_ref)

    sv = s0_ref[...] + s1_ref[...]
    h = jnp.maximum(_node_out(a0_ref[...] + a1_ref[...], sv, sk_ref[...], i),
                    0.0)
    rows = i * 128 + lax.broadcasted_iota(jnp.int32, (128, 1), 0)
    h = jnp.where(rows < N, h, 0.0)
    bvec = batch_ref[0, 0, :]
    seg = lax.broadcasted_iota(jnp.int32, (B, 128), 0)
    oh = jnp.where(seg == bvec[None, :], 1.0, 0.0)
    pooled_ref[...] += jnp.dot(oh, h, preferred_element_type=jnp.float32)
    cnt_ref[...] += jnp.dot(oh, jnp.ones((128, 128), jnp.float32),
                            preferred_element_type=jnp.float32)

    @pl.when(i == NBLOCKS - 1)
    def _():
        pooled = pooled_ref[...] / jnp.maximum(cnt_ref[...], 1.0)
        out_ref[...] = jnp.dot(pooled, wlt_ref[...],
                               preferred_element_type=jnp.float32) + bl_ref[...]


def _fin(a0, a1, s0, s1, sk, batch3d, wlt, bl):
    return pl.pallas_call(
        _fin_kernel,
        grid=(NBLOCKS,),
        in_specs=[
            pl.BlockSpec((128, H), lambda i: (i, 0)),
            pl.BlockSpec((128, H), lambda i: (i, 0)),
            pl.BlockSpec((128, 128), lambda i: (0, 0)),
            pl.BlockSpec((128, 128), lambda i: (0, 0)),
            pl.BlockSpec((128, H), lambda i: (i, 0)),
            pl.BlockSpec((1, 1, 128), lambda i: (i, 0, 0)),
            pl.BlockSpec((H, H), lambda i: (0, 0)),
            pl.BlockSpec((1, H), lambda i: (0, 0)),
        ],
        out_specs=pl.BlockSpec((B, H), lambda i: (0, 0)),
        out_shape=jax.ShapeDtypeStruct((B, H), jnp.float32),
        scratch_shapes=[
            pltpu.VMEM((B, H), jnp.float32),
            pltpu.VMEM((B, H), jnp.float32),
        ],
    )(a0, a1, s0, s1, sk, batch3d, wlt, bl)


_EDGE_KW = dict(
    out_type=(
        jax.ShapeDtypeStruct((NC, NP, H), jnp.float32),
        jax.ShapeDtypeStruct((NC, NS_ROWS, 128), jnp.float32),
    ),
    mesh=plsc.VectorSubcoreMesh(core_axis_name="c", subcore_axis_name="s"),
    compiler_params=pltpu.CompilerParams(needs_layout_passes=False),
    scratch_types=[
        pltpu.VMEM((3 * G,), jnp.int32),
        pltpu.VMEM((3 * G,), jnp.int32),
        pltpu.VMEM((G,), jnp.int32),
        pltpu.VMEM((G,), jnp.int32),
        pltpu.VMEM((G,), jnp.int32),
        pltpu.VMEM((G,), jnp.int32),
        pltpu.VMEM((G,), jnp.int32),
        pltpu.VMEM((G,), jnp.int32),
        pltpu.VMEM((G,), jnp.float32),
        pltpu.VMEM((G,), jnp.float32),
        pltpu.VMEM((H,), jnp.float32),
        pltpu.VMEM((T2W,), jnp.int32),
        pltpu.VMEM((G, H), jnp.float32),
        pltpu.VMEM((G, H), jnp.float32),
        pltpu.VMEM((G, H), jnp.float32),
        pltpu.VMEM((G, H), jnp.float32),
        pltpu.VMEM_SHARED((N, H), jnp.float32),
        pltpu.VMEM_SHARED((NS_ROWS, 128), jnp.float32),
        pltpu.SemaphoreType.DMA,
        pltpu.SemaphoreType.DMA,
        pltpu.SemaphoreType.DMA,
        pltpu.SemaphoreType.DMA,
        pltpu.SemaphoreType.DMA,
    ],
)


def _edge_body(q_hbm, k_hbm, v_hbm, t2_hbm, ed_hbm, wec_hbm,
               zero_hbm, outv_hbm, outs_hbm,
               ed0, ed1, dstb0, dsb0, dstb1, dsb1, cvb, srcvb, exb, exwb,
               wecb, t2b, qrows, krows, vrows, denrows, accv, accs,
               semq, semk, semv, semi, sems):
    cid = lax.axis_index("c")
    sid = lax.axis_index("s")
    lanes = jnp.arange(LN, dtype=jnp.int32)
    zvec = jnp.zeros((LN,), jnp.float32)
    pltpu.sync_copy(wec_hbm, wecb)
    pltpu.sync_copy(t2_hbm, t2b)
    wvs = [wecb[pl.ds(r * LN, LN)] for r in range(8)]

    @pl.when(sid < NS - 1)
    def _():
        pltpu.sync_copy(zero_hbm, accv.at[pl.ds(sid * RPT, RPT)])

    @pl.when(sid == NS - 1)
    def _():
        pltpu.sync_copy(zero_hbm.at[pl.ds(0, N - (NS - 1) * RPT)],
                        accv.at[pl.ds((NS - 1) * RPT, N - (NS - 1) * RPT)])

    @pl.when(sid < NS_ROWS // 16)
    def _():
        pltpu.sync_copy(zero_hbm.at[pl.ds(0, 16)],
                        accs.at[pl.ds(sid * 16, 16)])

    def zrow(i, carry):
        for r in range(8):
            denrows[i, pl.ds(r * LN, LN)] = zvec
        return carry

    lax.fori_loop(0, G, zrow, 0)
    plsc.subcore_barrier()

    wid = cid * NS + sid
    gb0 = wid * NBLK

    def do_block(bi, edc, edn, dstb, dsb):
        pltpu.make_async_copy(zero_hbm.at[pl.ds(0, G)], qrows, semq).wait()
        pltpu.make_async_copy(zero_hbm.at[pl.ds(0, G)], krows, semk).wait()

        for g in range(G // LN):
            jv = lanes + (g * LN)
            dv16 = edc[pl.ds(G + g * LN, LN)]
            valid = dv16 < N
            dvc = jnp.where(valid, dv16, 0)

            def dotstep(dd, acc16):
                dv = jnp.zeros((LN,), jnp.int32) + dd
                qv = plsc.load_gather(qrows, [jv, dv])
                kv = plsc.load_gather(krows, [jv, dv])
                return acc16 + qv * kv

            dot = lax.fori_loop(0, H, dotstep, jnp.zeros((LN,), jnp.float32),
                                unroll=4)
            w = plsc.load_gather(t2b, [dvc >> 1])
            bits = jnp.where((dvc & 1) == 0, w << 16, w & jnp.int32(-65536))
            tv = plsc.bitcast(bits, jnp.float32)
            ewv = plsc.bitcast(edc[pl.ds(2 * G + g * LN, LN)], jnp.float32)
            ex = jnp.where(valid, jnp.exp(dot + ewv * tv), 0.0)
            exb[pl.ds(g * LN, LN)] = ex
            exwb[pl.ds(g * LN, LN)] = ex * ewv
            dstb[pl.ds(g * LN, LN)] = dvc
            srcvb[pl.ds(g * LN, LN)] = edc[pl.ds(g * LN, LN)]

        @pl.when(bi < NBLK - 1)
        def _():
            pltpu.make_async_copy(ed_hbm.at[pl.ds(0, 3 * G)], edn,
                                  semi).wait()
            pltpu.async_copy(q_hbm.at[edn.at[pl.ds(G, G)]], qrows, semq)
            pltpu.async_copy(k_hbm.at[edn.at[pl.ds(0, G)]], krows, semk)

        @pl.when(bi < NBLK - 2)
        def _():
            pltpu.async_copy(
                ed_hbm.at[pl.ds((gb0 + bi + 2) * (3 * G), 3 * G)], edc, semi)

        @pl.when(bi > 0)
        def _():
            pltpu.make_async_copy(zero_hbm.at[pl.ds(0, G)], vrows,
                                  sems).wait()
            pltpu.make_async_copy(zero_hbm.at[pl.ds(0, G)], denrows,
                                  sems).wait()

        cpv = pltpu.async_copy(v_hbm.at[srcvb], vrows, semv)

        @pl.when(bi > 0)
        def _():
            for g in range(G // LN):
                jv = lanes + (g * LN)
                plsc.store_scatter(denrows, [jv, cvb[pl.ds(g * LN, LN)]],
                                   zvec)

        for g in range(G // LN):
            jv = lanes + (g * LN)
            dvc = dstb[pl.ds(g * LN, LN)]
            dsb[pl.ds(g * LN, LN)] = dvc & 127
            cvb[pl.ds(g * LN, LN)] = dvc >> 7
            plsc.store_scatter(denrows, [jv, dvc >> 7],
                               exb[pl.ds(g * LN, LN)])

        cpv.wait()
        for g in range(G // LN):
            ex = exb[pl.ds(g * LN, LN)]
            exw = exwb[pl.ds(g * LN, LN)]
            for j in range(LN):
                jj = g * LN + j
                exj = lax.index_in_dim(ex, j, keepdims=False)
                exwj = lax.index_in_dim(exw, j, keepdims=False)
                for r in range(8):
                    vrows[jj, pl.ds(r * LN, LN)] = (
                        vrows[jj, pl.ds(r * LN, LN)] * exj + wvs[r] * exwj)
        pltpu.async_copy(vrows, accv.at[dstb], sems, add=True)
        pltpu.async_copy(denrows, accs.at[dsb], sems, add=True)

    pltpu.sync_copy(ed_hbm.at[pl.ds(gb0 * (3 * G), 3 * G)], ed0)
    pltpu.async_copy(q_hbm.at[ed0.at[pl.ds(G, G)]], qrows, semq)
    pltpu.async_copy(k_hbm.at[ed0.at[pl.ds(0, G)]], krows, semk)
    pltpu.async_copy(ed_hbm.at[pl.ds((gb0 + 1) * (3 * G), 3 * G)], ed1, semi)

    def block_pair(pi, carry):
        do_block(2 * pi, ed0, ed1, dstb0, dsb0)
        do_block(2 * pi + 1, ed1, ed0, dstb1, dsb1)
        return carry

    lax.fori_loop(0, NBLK // 2, block_pair, 0)
    do_block(jnp.int32(NBLK - 1), ed0, ed1, dstb0, dsb0)
    pltpu.make_async_copy(zero_hbm.at[pl.ds(0, G)], vrows, sems).wait()
    pltpu.make_async_copy(zero_hbm.at[pl.ds(0, G)], denrows, sems).wait()
    plsc.subcore_barrier()

    @pl.when(sid < NS - 1)
    def _():
        pltpu.sync_copy(accv.at[pl.ds(sid * RPT, RPT)],
                        outv_hbm.at[cid, pl.ds(sid * RPT, RPT)])

    @pl.when(sid == NS - 1)
    def _():
        pltpu.sync_copy(accv.at[pl.ds((NS - 1) * RPT, N - (NS - 1) * RPT)],
                        outv_hbm.at[cid, pl.ds((NS - 1) * RPT,
                                               N - (NS - 1) * RPT)])

    @pl.when(sid < NS_ROWS // 16)
    def _():
        pltpu.sync_copy(accs.at[pl.ds(sid * 16, 16)],
                        outs_hbm.at[cid, pl.ds(sid * 16, 16)])


_edge_kernel = pl.kernel(_edge_body, **_EDGE_KW)


def _layer_weights(Wq, bq, Wk, bk, Wv, bv, We, Ws, bs):
    rs = 1.0 / jnp.sqrt(float(H))
    wec = We[:, 0]
    wt = (Wq.T @ wec) * rs
    bt = jnp.dot(bq, wec) * rs
    W = jnp.concatenate([
        Wq.T * rs, Wk.T, Wv.T, Ws.T,
        wt[:, None], jnp.zeros((H, 127), jnp.float32)], axis=1)
    b = jnp.concatenate([
        bq * rs, bk, bv, bs, bt[None], jnp.zeros((127,), jnp.float32)])
    return W, b[None, :], wec


def _pack_t(Tmat):
    tb = Tmat[:N, 0].astype(jnp.bfloat16)
    u16 = lax.bitcast_convert_type(tb, jnp.uint16).astype(jnp.uint32)
    pairs = u16.reshape(N // 2, 2)
    t2 = pairs[:, 0] | (pairs[:, 1] << 16)
    return jnp.pad(t2.astype(jnp.int32), (0, T2W - N // 2))


def kernel(x, edge_index, edge_weight, batch,
           Wq1, bq1, Wk1, bk1, Wv1, bv1, We1, Ws1, bs1,
           Wq2, bq2, Wk2, bk2, Wv2, bv2, We2, Ws2, bs2,
           Wl, bl):
    src = edge_index[0]
    dst = edge_index[1]
    ew = edge_weight

    W1, b1, wec1 = _layer_weights(Wq1, bq1, Wk1, bk1, Wv1, bv1, We1, Ws1, bs1)
    W2, b2, wec2 = _layer_weights(Wq2, bq2, Wk2, bk2, Wv2, bv2, We2, Ws2, bs2)
    zero = jnp.zeros((RPT, 128), jnp.float32)
    ewbits = lax.bitcast_convert_type(ew, jnp.int32)
    epad = NBLK * G - EPT
    srcp = jnp.pad(src.reshape(NTILES, EPT), ((0, 0), (0, epad)))
    dstp = jnp.pad(dst.reshape(NTILES, EPT), ((0, 0), (0, epad)),
                   constant_values=N)
    ewp = jnp.pad(ewbits.reshape(NTILES, EPT), ((0, 0), (0, epad)))
    ed = jnp.stack([srcp.reshape(NTILES, NBLK, G),
                    dstp.reshape(NTILES, NBLK, G),
                    ewp.reshape(NTILES, NBLK, G)], axis=2).reshape(-1)

    xp = jnp.pad(x, ((0, NP - N), (0, 0)))
    Q1, K1, V1, S1, T1 = _lift(xp, W1, b1)
    av1, as1 = _edge_kernel(Q1, K1, V1, _pack_t(T1), ed, wec1, zero)
    Q2, K2, V2, S2, T2 = _mid(av1[0], av1[1], as1[0], as1[1], S1, W2, b2)
    av2, as2 = _edge_kernel(Q2, K2, V2, _pack_t(T2), ed, wec2, zero)

    batch3d = jnp.pad(batch, (0, NP - N), constant_values=B).reshape(
        NBLOCKS, 1, 128)
    wlt = jnp.zeros((H, H), jnp.float32).at[:, :OUT].set(Wl.T)
    blp = jnp.zeros((1, H), jnp.float32).at[0, :OUT].set(bl)
    out = _fin(av2[0], av2[1], as2[0], as2[1], S2, batch3d, wlt, blp)
    return out[:, :OUT]


# speedup vs baseline: 1.0402x; 1.0402x over previous
import jax
import jax.numpy as jnp
from jax import lax
from jax.experimental import pallas as pl
from jax.experimental.pallas import tpu as pltpu
from jax.experimental.pallas import tpu_sc as plsc

N = 10000
E = 320000
H = 128
B = 64
OUT = 10
NP = 10112
NBLOCKS = NP // 128

NC = 2
NS = 16
LN = 16
NTILES = NC * NS
EPT = E // NTILES
G = 80
NBLK = EPT // G
RPT = 632
NS_ROWS = 128
T2W = 5008


def _lift_kernel(x_ref, w_ref, b_ref, q_ref, k_ref, v_ref, s_ref, t_ref):
    y = jnp.dot(x_ref[...], w_ref[...], preferred_element_type=jnp.float32)
    y = y + b_ref[...]
    q_ref[...] = y[:, :128]
    k_ref[...] = y[:, 128:256]
    v_ref[...] = y[:, 256:384]
    s_ref[...] = y[:, 384:512]
    t_ref[...] = y[:, 512:]


_LIFT_OUT_SPECS = [
    pl.BlockSpec((128, H), lambda i: (i, 0)),
    pl.BlockSpec((128, H), lambda i: (i, 0)),
    pl.BlockSpec((128, H), lambda i: (i, 0)),
    pl.BlockSpec((128, H), lambda i: (i, 0)),
    pl.BlockSpec((128, H), lambda i: (i, 0)),
]
_LIFT_OUT_SHAPE = [
    jax.ShapeDtypeStruct((NP, H), jnp.float32),
    jax.ShapeDtypeStruct((NP, H), jnp.float32),
    jax.ShapeDtypeStruct((NP, H), jnp.float32),
    jax.ShapeDtypeStruct((NP, H), jnp.float32),
    jax.ShapeDtypeStruct((NP, H), jnp.float32),
]


def _lift(xp, W, b):
    return pl.pallas_call(
        _lift_kernel,
        grid=(NBLOCKS,),
        in_specs=[
            pl.BlockSpec((128, H), lambda i: (i, 0)),
            pl.BlockSpec((H, 640), lambda i: (0, 0)),
            pl.BlockSpec((1, 640), lambda i: (0, 0)),
        ],
        out_specs=_LIFT_OUT_SPECS,
        out_shape=_LIFT_OUT_SHAPE,
    )(xp, W, b)


def _node_out(av, sv, s, i):
    ohi = jnp.where(
        lax.broadcasted_iota(jnp.int32, (128, 128), 0) == i, 1.0, 0.0)
    den = jnp.dot(sv, ohi, preferred_element_type=jnp.float32)
    return av / (den + 1e-16) + s


def _mid_kernel(a0_ref, a1_ref, s0_ref, s1_ref, sk_ref, w_ref, b_ref,
                q_ref, k_ref, v_ref, s2_ref, t_ref):
    i = pl.program_id(0)
    sv = s0_ref[...] + s1_ref[...]
    h = jnp.maximum(_node_out(a0_ref[...] + a1_ref[...], sv, sk_ref[...], i),
                    0.0)
    rows = i * 128 + lax.broadcasted_iota(jnp.int32, (128, 1), 0)
    h = jnp.where(rows < N, h, 0.0)
    y = jnp.dot(h, w_ref[...], preferred_element_type=jnp.float32)
    y = y + b_ref[...]
    q_ref[...] = y[:, :128]
    k_ref[...] = y[:, 128:256]
    v_ref[...] = y[:, 256:384]
    s2_ref[...] = y[:, 384:512]
    t_ref[...] = y[:, 512:]


def _mid(a0, a1, s0, s1, sk, W, b):
    return pl.pallas_call(
        _mid_kernel,
        grid=(NBLOCKS,),
        in_specs=[
            pl.BlockSpec((128, H), lambda i: (i, 0)),
            pl.BlockSpec((128, H), lambda i: (i, 0)),
            pl.BlockSpec((128, 128), lambda i: (0, 0)),
            pl.BlockSpec((128, 128), lambda i: (0, 0)),
            pl.BlockSpec((128, H), lambda i: (i, 0)),
            pl.BlockSpec((H, 640), lambda i: (0, 0)),
            pl.BlockSpec((1, 640), lambda i: (0, 0)),
        ],
        out_specs=_LIFT_OUT_SPECS,
        out_shape=_LIFT_OUT_SHAPE,
    )(a0, a1, s0, s1, sk, W, b)


def _fin_kernel(a0_ref, a1_ref, s0_ref, s1_ref, sk_ref, batch_ref,
                wlt_ref, bl_ref, out_ref, pooled_ref, cnt_ref):
    i = pl.program_id(0)

    @pl.when(i == 0)
    def _():
        pooled_ref[...] = jnp.zeros_like(pooled_ref)
        cnt_ref[...] = jnp.zeros_like(cnt_ref)

    sv = s0_ref[...] + s1_ref[...]
    h = jnp.maximum(_node_out(a0_ref[...] + a1_ref[...], sv, sk_ref[...], i),
                    0.0)
    rows = i * 128 + lax.broadcasted_iota(jnp.int32, (128, 1), 0)
    h = jnp.where(rows < N, h, 0.0)
    bvec = batch_ref[0, 0, :]
    seg = lax.broadcasted_iota(jnp.int32, (B, 128), 0)
    oh = jnp.where(seg == bvec[None, :], 1.0, 0.0)
    pooled_ref[...] += jnp.dot(oh, h, preferred_element_type=jnp.float32)
    cnt_ref[...] += jnp.dot(oh, jnp.ones((128, 128), jnp.float32),
                            preferred_element_type=jnp.float32)

    @pl.when(i == NBLOCKS - 1)
    def _():
        pooled = pooled_ref[...] / jnp.maximum(cnt_ref[...], 1.0)
        out_ref[...] = jnp.dot(pooled, wlt_ref[...],
                               preferred_element_type=jnp.float32) + bl_ref[...]


def _fin(a0, a1, s0, s1, sk, batch3d, wlt, bl):
    return pl.pallas_call(
        _fin_kernel,
        grid=(NBLOCKS,),
        in_specs=[
            pl.BlockSpec((128, H), lambda i: (i, 0)),
            pl.BlockSpec((128, H), lambda i: (i, 0)),
            pl.BlockSpec((128, 128), lambda i: (0, 0)),
            pl.BlockSpec((128, 128), lambda i: (0, 0)),
            pl.BlockSpec((128, H), lambda i: (i, 0)),
            pl.BlockSpec((1, 1, 128), lambda i: (i, 0, 0)),
            pl.BlockSpec((H, H), lambda i: (0, 0)),
            pl.BlockSpec((1, H), lambda i: (0, 0)),
        ],
        out_specs=pl.BlockSpec((B, H), lambda i: (0, 0)),
        out_shape=jax.ShapeDtypeStruct((B, H), jnp.float32),
        scratch_shapes=[
            pltpu.VMEM((B, H), jnp.float32),
            pltpu.VMEM((B, H), jnp.float32),
        ],
    )(a0, a1, s0, s1, sk, batch3d, wlt, bl)


_EDGE_KW = dict(
    out_type=(
        jax.ShapeDtypeStruct((NC, NP, H), jnp.float32),
        jax.ShapeDtypeStruct((NC, NS_ROWS, 128), jnp.float32),
    ),
    mesh=plsc.VectorSubcoreMesh(core_axis_name="c", subcore_axis_name="s"),
    compiler_params=pltpu.CompilerParams(needs_layout_passes=False),
    scratch_types=[
        pltpu.VMEM((3 * G,), jnp.int32),
        pltpu.VMEM((3 * G,), jnp.int32),
        pltpu.VMEM((G,), jnp.int32),
        pltpu.VMEM((G,), jnp.int32),
        pltpu.VMEM((G,), jnp.int32),
        pltpu.VMEM((H,), jnp.float32),
        pltpu.VMEM((T2W,), jnp.int32),
        pltpu.VMEM((G, H), jnp.float32),
        pltpu.VMEM((G, H), jnp.float32),
        pltpu.VMEM((G, H), jnp.float32),
        pltpu.VMEM((G, H), jnp.float32),
        pltpu.VMEM_SHARED((N, H), jnp.float32),
        pltpu.VMEM_SHARED((NS_ROWS, 128), jnp.float32),
        pltpu.SemaphoreType.DMA,
        pltpu.SemaphoreType.DMA,
        pltpu.SemaphoreType.DMA,
        pltpu.SemaphoreType.DMA,
        pltpu.SemaphoreType.DMA,
    ],
)


def _edge_body(q_hbm, k_hbm, v_hbm, t2_hbm, ed_hbm, wec_hbm,
               zero_hbm, outv_hbm, outs_hbm,
               ed0, ed1, dstb, dsb, cvb, wecb, t2b, qrows, krows, vrows,
               denrows, accv, accs, semq, semk, semv, semi, sems):
    cid = lax.axis_index("c")
    sid = lax.axis_index("s")
    lanes = jnp.arange(LN, dtype=jnp.int32)
    zvec = jnp.zeros((LN,), jnp.float32)
    pltpu.sync_copy(wec_hbm, wecb)
    pltpu.sync_copy(t2_hbm, t2b)
    wvs = [wecb[pl.ds(r * LN, LN)] for r in range(8)]

    @pl.when(sid < NS - 1)
    def _():
        pltpu.sync_copy(zero_hbm, accv.at[pl.ds(sid * RPT, RPT)])

    @pl.when(sid == NS - 1)
    def _():
        pltpu.sync_copy(zero_hbm.at[pl.ds(0, N - (NS - 1) * RPT)],
                        accv.at[pl.ds((NS - 1) * RPT, N - (NS - 1) * RPT)])

    @pl.when(sid < NS_ROWS // 16)
    def _():
        pltpu.sync_copy(zero_hbm.at[pl.ds(0, 16)],
                        accs.at[pl.ds(sid * 16, 16)])

    def zrow(i, carry):
        for r in range(8):
            denrows[i, pl.ds(r * LN, LN)] = zvec
        return carry

    lax.fori_loop(0, G, zrow, 0)
    plsc.subcore_barrier()

    wid = cid * NS + sid
    gb0 = wid * NBLK

    def do_block(bi, edc, edn):
        @pl.when(bi > 0)
        def _():
            pltpu.make_async_copy(ed_hbm.at[pl.ds(0, 3 * G)], edc,
                                  semi).wait()

        cpq = pltpu.async_copy(q_hbm.at[edc.at[pl.ds(G, G)]], qrows, semq)
        cpk = pltpu.async_copy(k_hbm.at[edc.at[pl.ds(0, G)]], krows, semk)

        @pl.when(bi > 0)
        def _():
            pltpu.make_async_copy(zero_hbm.at[pl.ds(0, G)], vrows,
                                  sems).wait()
            pltpu.make_async_copy(zero_hbm.at[pl.ds(0, G)], denrows,
                                  sems).wait()

        cpv = pltpu.async_copy(v_hbm.at[edc.at[pl.ds(0, G)]], vrows, semv)

        @pl.when(bi < NBLK - 1)
        def _():
            pltpu.async_copy(
                ed_hbm.at[pl.ds((gb0 + bi + 1) * (3 * G), 3 * G)], edn, semi)

        @pl.when(bi > 0)
        def _():
            for g in range(G // LN):
                jv = lanes + (g * LN)
                plsc.store_scatter(denrows, [jv, cvb[pl.ds(g * LN, LN)]],
                                   zvec)

        cpq.wait()
        cpk.wait()
        cpv.wait()
        for g in range(G // LN):
            jv = lanes + (g * LN)
            dv16 = edc[pl.ds(G + g * LN, LN)]

            def dotstep(dd, acc16):
                dv = jnp.zeros((LN,), jnp.int32) + dd
                qv = plsc.load_gather(qrows, [jv, dv])
                kv = plsc.load_gather(krows, [jv, dv])
                return acc16 + qv * kv

            dot = lax.fori_loop(0, H, dotstep, jnp.zeros((LN,), jnp.float32),
                                unroll=4)
            w = plsc.load_gather(t2b, [dv16 >> 1])
            bits = jnp.where((dv16 & 1) == 0, w << 16, w & jnp.int32(-65536))
            tv = plsc.bitcast(bits, jnp.float32)
            ewv = plsc.bitcast(edc[pl.ds(2 * G + g * LN, LN)], jnp.float32)
            ex = jnp.exp(dot + ewv * tv)
            exw = ex * ewv
            dstb[pl.ds(g * LN, LN)] = dv16
            dsb[pl.ds(g * LN, LN)] = dv16 & 127
            cvb[pl.ds(g * LN, LN)] = dv16 >> 7
            plsc.store_scatter(denrows, [jv, dv16 >> 7], ex)
            for j in range(LN):
                jj = g * LN + j
                exj = lax.index_in_dim(ex, j, keepdims=False)
                exwj = lax.index_in_dim(exw, j, keepdims=False)
                for r in range(8):
                    vrows[jj, pl.ds(r * LN, LN)] = (
                        vrows[jj, pl.ds(r * LN, LN)] * exj + wvs[r] * exwj)
        pltpu.async_copy(vrows, accv.at[dstb], sems, add=True)
        pltpu.async_copy(denrows, accs.at[dsb], sems, add=True)

    pltpu.sync_copy(ed_hbm.at[pl.ds(gb0 * (3 * G), 3 * G)], ed0)

    def block_pair(pi, carry):
        do_block(2 * pi, ed0, ed1)
        do_block(2 * pi + 1, ed1, ed0)
        return carry

    lax.fori_loop(0, NBLK // 2, block_pair, 0)
    do_block(jnp.int32(NBLK - 1), ed0, ed1)
    pltpu.make_async_copy(zero_hbm.at[pl.ds(0, G)], vrows, sems).wait()
    pltpu.make_async_copy(zero_hbm.at[pl.ds(0, G)], denrows, sems).wait()
    plsc.subcore_barrier()

    @pl.when(sid < NS - 1)
    def _():
        pltpu.sync_copy(accv.at[pl.ds(sid * RPT, RPT)],
                        outv_hbm.at[cid, pl.ds(sid * RPT, RPT)])

    @pl.when(sid == NS - 1)
    def _():
        pltpu.sync_copy(accv.at[pl.ds((NS - 1) * RPT, N - (NS - 1) * RPT)],
                        outv_hbm.at[cid, pl.ds((NS - 1) * RPT,
                                               N - (NS - 1) * RPT)])

    @pl.when(sid < NS_ROWS // 16)
    def _():
        pltpu.sync_copy(accs.at[pl.ds(sid * 16, 16)],
                        outs_hbm.at[cid, pl.ds(sid * 16, 16)])


_edge_kernel = pl.kernel(_edge_body, **_EDGE_KW)


def _layer_weights(Wq, bq, Wk, bk, Wv, bv, We, Ws, bs):
    rs = 1.0 / jnp.sqrt(float(H))
    wec = We[:, 0]
    wt = (Wq.T @ wec) * rs
    bt = jnp.dot(bq, wec) * rs
    W = jnp.concatenate([
        Wq.T * rs, Wk.T, Wv.T, Ws.T,
        wt[:, None], jnp.zeros((H, 127), jnp.float32)], axis=1)
    b = jnp.concatenate([
        bq * rs, bk, bv, bs, bt[None], jnp.zeros((127,), jnp.float32)])
    return W, b[None, :], wec


def _pack_t(Tmat):
    tb = Tmat[:N, 0].astype(jnp.bfloat16)
    u16 = lax.bitcast_convert_type(tb, jnp.uint16).astype(jnp.uint32)
    pairs = u16.reshape(N // 2, 2)
    t2 = pairs[:, 0] | (pairs[:, 1] << 16)
    return jnp.pad(t2.astype(jnp.int32), (0, T2W - N // 2))


def kernel(x, edge_index, edge_weight, batch,
           Wq1, bq1, Wk1, bk1, Wv1, bv1, We1, Ws1, bs1,
           Wq2, bq2, Wk2, bk2, Wv2, bv2, We2, Ws2, bs2,
           Wl, bl):
    src = edge_index[0]
    dst = edge_index[1]
    ew = edge_weight

    W1, b1, wec1 = _layer_weights(Wq1, bq1, Wk1, bk1, Wv1, bv1, We1, Ws1, bs1)
    W2, b2, wec2 = _layer_weights(Wq2, bq2, Wk2, bk2, Wv2, bv2, We2, Ws2, bs2)
    zero = jnp.zeros((RPT, 128), jnp.float32)
    ewbits = lax.bitcast_convert_type(ew, jnp.int32)
    ed = jnp.stack([src.reshape(-1, G), dst.reshape(-1, G),
                    ewbits.reshape(-1, G)], axis=1).reshape(-1)

    xp = jnp.pad(x, ((0, NP - N), (0, 0)))
    Q1, K1, V1, S1, T1 = _lift(xp, W1, b1)
    av1, as1 = _edge_kernel(Q1, K1, V1, _pack_t(T1), ed, wec1, zero)
    Q2, K2, V2, S2, T2 = _mid(av1[0], av1[1], as1[0], as1[1], S1, W2, b2)
    av2, as2 = _edge_kernel(Q2, K2, V2, _pack_t(T2), ed, wec2, zero)

    batch3d = jnp.pad(batch, (0, NP - N), constant_values=B).reshape(
        NBLOCKS, 1, 128)
    wlt = jnp.zeros((H, H), jnp.float32).at[:, :OUT].set(Wl.T)
    blp = jnp.zeros((1, H), jnp.float32).at[0, :OUT].set(bl)
    out = _fin(av2[0], av2[1], as2[0], as2[1], S2, batch3d, wlt, blp)
    return out[:, :OUT]
